# baseline (speedup 1.0000x reference)
import functools

import jax
import jax.numpy as jnp
from jax import lax
from jax.experimental import pallas as pl
from jax.experimental.pallas import tpu as pltpu
from jax.experimental.pallas import tpu_sc as plsc

N = 10000
K = 16
L = 2
H = 128
NG = 20
NE = N * K
RB = 400
NBLK = N // RB
EB = RB * K
KRB = 200
KNBLK = N // KRB
CB = 256
NPAD = 10624

_OFF = jnp.linspace(0.0, 10.0, NG)
_COEFF = -0.5 / (float(_OFF[1] - _OFF[0]) ** 2)

_BIGI = 0x3FFFFFFF


def _knn_body(rng_ref, a_ref, b_ref, bcol_ref, nbr_ref):
    i = pl.program_id(0)
    lo = rng_ref[i, 0]
    hi = rng_ref[i, 1]
    ablk = a_ref[...]
    brow = ablk[:, 5:6]
    sqr = ablk[:, 3:4]
    rowg = lax.broadcasted_iota(jnp.int32, (KRB, 1), 0) + i * KRB
    ntiles = (hi - lo + CB - 1) // CB

    def tile_step(t, carry):
        bd, bi = carry
        start = pl.multiple_of(lo + t * CB, 128)
        bt = b_ref[:, pl.ds(start, CB)]
        bct = bcol_ref[:, pl.ds(start, CB)]
        sqc = bt[4:5, :]
        cross = jnp.dot(ablk, bt, preferred_element_type=jnp.float32)
        d = (sqr + sqc) - cross
        colg = lax.broadcasted_iota(jnp.int32, (KRB, CB), 1) + start
        valid = (brow == bct) & (colg != rowg)
        d = jnp.where(valid, d, jnp.inf)
        lane_k = lax.broadcasted_iota(jnp.int32, (KRB, K), 1)
        nbd = jnp.full((KRB, K), jnp.inf, jnp.float32)
        nbi = jnp.full((KRB, K), _BIGI, jnp.int32)
        for k in range(K):
            m = jnp.minimum(jnp.min(d, axis=1, keepdims=True),
                            jnp.min(bd, axis=1, keepdims=True))
            ct = jnp.min(jnp.where(d <= m, colg, _BIGI), axis=1, keepdims=True)
            cb = jnp.min(jnp.where(bd <= m, bi, _BIGI), axis=1, keepdims=True)
            amin = jnp.minimum(ct, cb)
            d = jnp.where(colg == amin, jnp.inf, d)
            bd = jnp.where(bi == amin, jnp.inf, bd)
            nbd = jnp.where(lane_k == k, m, nbd)
            nbi = jnp.where(lane_k == k, amin, nbi)
        return nbd, nbi

    bd0 = jnp.full((KRB, K), jnp.inf, jnp.float32)
    bi0 = jnp.full((KRB, K), _BIGI, jnp.int32)
    _, bi = lax.fori_loop(0, ntiles, tile_step, (bd0, bi0))
    nbr_ref[...] = bi


def _knn_call(rng, a, b, bcol):
    return pl.pallas_call(
        _knn_body,
        grid=(KNBLK,),
        in_specs=[
            pl.BlockSpec(memory_space=pltpu.SMEM),
            pl.BlockSpec((KRB, 8), lambda i: (i, 0)),
            pl.BlockSpec((8, NPAD), lambda i: (0, 0)),
            pl.BlockSpec((1, NPAD), lambda i: (0, 0)),
        ],
        out_specs=pl.BlockSpec((KRB, K), lambda i: (i, 0)),
        out_shape=jax.ShapeDtypeStruct((N, K), jnp.int32),
        compiler_params=pltpu.CompilerParams(
            dimension_semantics=("parallel",)),
    )(rng, a, b, bcol)


_NW = 32
_PW = NE // _NW
_CH = 104
_NCHT = 49
_NG16 = _PW // 16


def _gather_body(h_hbm, idx_hbm, xm_hbm, outh_hbm, outx_hbm,
                 idx_all, xm4, xmout, buf0, buf1, semi, sem0, sem1):
    wid = lax.axis_index("s") * 2 + lax.axis_index("c")
    base = wid * _PW
    pltpu.async_copy(idx_hbm.at[pl.ds(base, _PW)], idx_all, semi).wait()

    def off_of(c):
        return jnp.minimum(c * _CH, _PW - _CH)

    def issue(c, buf, sem):
        pltpu.async_copy(h_hbm.at[idx_all.at[pl.ds(off_of(c), _CH)]], buf, sem)

    def wait(buf, sem):
        pltpu.make_async_copy(h_hbm.at[pl.ds(0, _CH)], buf, sem).wait()

    def out(c, buf):
        pltpu.sync_copy(buf, outh_hbm.at[pl.ds(base + off_of(c), _CH)])

    issue(0, buf0, sem0)
    issue(1, buf1, sem1)

    pltpu.async_copy(xm_hbm, xm4, semi).wait()
    i4 = lax.iota(jnp.int32, 16) * 4

    def xstep(g, carry):
        goff = jnp.minimum(g * 16, _PW - 16)
        iv = idx_all[pl.ds(goff, 16)] * 4
        pos = i4 + goff * 4
        for c in range(4):
            v = plsc.load_gather(xm4, [iv + c])
            plsc.store_scatter(xmout, [pos + c], v)
        return carry

    lax.fori_loop(0, _NG16 + 1, xstep, 0)
    pltpu.sync_copy(xmout, outx_hbm.at[pl.ds(base * 4, _PW * 4)])

    def pair(cc, carry):
        c0 = 2 * cc
        wait(buf0, sem0)
        out(c0, buf0)
        issue(jnp.minimum(c0 + 2, _NCHT - 1), buf0, sem0)
        wait(buf1, sem1)
        out(c0 + 1, buf1)
        issue(jnp.minimum(c0 + 3, _NCHT - 1), buf1, sem1)
        return carry

    lax.fori_loop(0, 25, pair, 0)
    wait(buf0, sem0)
    wait(buf1, sem1)


def _gather_call(h, idx, xm4flat):
    mesh = plsc.VectorSubcoreMesh(
        core_axis_name="c", subcore_axis_name="s", num_cores=2, num_subcores=16)
    fn = pl.kernel(
        _gather_body,
        out_type=(jax.ShapeDtypeStruct((NE, H), jnp.float32),
                  jax.ShapeDtypeStruct((NE * 4,), jnp.float32)),
        mesh=mesh,
        compiler_params=pltpu.CompilerParams(needs_layout_passes=False),
        scratch_types=[
            pltpu.VMEM((_PW,), jnp.int32),
            pltpu.VMEM((N * 4,), jnp.float32),
            pltpu.VMEM((_PW * 4,), jnp.float32),
            pltpu.VMEM((_CH, H), jnp.float32),
            pltpu.VMEM((_CH, H), jnp.float32),
            pltpu.SemaphoreType.DMA,
            pltpu.SemaphoreType.DMA,
            pltpu.SemaphoreType.DMA,
        ],
    )
    return fn(h, idx, xm4flat)


def _bcast_e(v):
    c = v.shape[1]
    return jnp.broadcast_to(v[:, None, :], (RB, K, c)).reshape(EB, c)


def _segsum(v):
    c = v.shape[1]
    return jnp.sum(v.reshape(RB, K, c), axis=1)


def _edge_body(gh_ref, gx_ref, h_ref, xp_ref,
               w1a_ref, w1b_ref, w1c_ref, w1d_ref, b1_ref,
               w2_ref, b2_ref, winf_ref, binf_ref,
               wx1_ref, bx1_ref, wx2_ref,
               wn1a_ref, wn1b_ref, bn1_ref, wn2_ref, bn2_ref,
               offs_ref,
               h2_ref, x2_ref):
    f32 = jnp.float32
    hi_blk = h_ref[...]
    xi4 = xp_ref[...]
    hj = gh_ref[...]
    tj = gx_ref[...]

    lane4e = lax.broadcasted_iota(jnp.int32, (EB, 4), 1)

    xie = _bcast_e(xi4)
    rel = jnp.where(lane4e < 3, xie - tj, 0.0)
    d_sq = jnp.sum(rel * rel, axis=1, keepdims=True)
    dd = jnp.sqrt(d_sq + 1e-8)
    offs = offs_ref[...]
    dfeat = jnp.exp(_COEFF * (dd - offs) ** 2)

    msrc = tj[:, 3:4]
    mdst_n = xi4[:, 3:4]
    mdst = _bcast_e(mdst_n)
    lane8 = lax.broadcasted_iota(jnp.int32, (EB, 8), 1)
    ea = ((lane8 == 0).astype(f32) * (msrc * mdst)
          + (lane8 == 1).astype(f32) * (msrc * (1.0 - mdst))
          + (lane8 == 2).astype(f32) * ((1.0 - msrc) * mdst)
          + (lane8 == 3).astype(f32) * ((1.0 - msrc) * (1.0 - mdst)))

    pre_i = jnp.dot(hi_blk, w1a_ref[...], preferred_element_type=f32) + b1_ref[...]
    pre = (_bcast_e(pre_i)
           + jnp.dot(hj, w1b_ref[...], preferred_element_type=f32)
           + jnp.dot(dfeat, w1c_ref[...], preferred_element_type=f32)
           + jnp.dot(ea, w1d_ref[...], preferred_element_type=f32))
    t1 = pre * jax.nn.sigmoid(pre)
    t2 = jnp.dot(t1, w2_ref[...], preferred_element_type=f32) + b2_ref[...]
    mij = t2 * jax.nn.sigmoid(t2)

    zinf = jnp.dot(mij, winf_ref[...], preferred_element_type=f32) + binf_ref[...]
    zinf0 = jnp.sum(jnp.where(lane8 == 0, zinf, 0.0), axis=1, keepdims=True)
    eij = jax.nn.sigmoid(zinf0)

    mi = _segsum(mij * eij)
    u = (jnp.dot(mi, wn1a_ref[...], preferred_element_type=f32)
         + jnp.dot(hi_blk, wn1b_ref[...], preferred_element_type=f32)
         + bn1_ref[...])
    u = u * jax.nn.sigmoid(u)
    h2_ref[...] = hi_blk + jnp.dot(u, wn2_ref[...], preferred_element_type=f32) + bn2_ref[...]

    s = jnp.dot(mij, wx1_ref[...], preferred_element_type=f32) + bx1_ref[...]
    s = s * jax.nn.sigmoid(s)
    zx = jnp.dot(s, wx2_ref[...], preferred_element_type=f32)
    zx0 = jnp.sum(jnp.where(lane8 == 0, zx, 0.0), axis=1, keepdims=True)
    xm = jnp.tanh(zx0)
    delta = rel * (xm / (dd + 1.0))
    dx = _segsum(delta)
    x2_ref[...] = xi4 + dx * mdst_n


def _edge_call(gh, gx, h, xp, wts, offs):
    full = lambda shape: pl.BlockSpec(shape, lambda i: (0, 0))
    return pl.pallas_call(
        _edge_body,
        grid=(NBLK,),
        in_specs=[
            pl.BlockSpec((EB, H), lambda i: (i, 0)),
            pl.BlockSpec((EB, 4), lambda i: (i, 0)),
            pl.BlockSpec((RB, H), lambda i: (i, 0)),
            pl.BlockSpec((RB, 4), lambda i: (i, 0)),
            full((H, H)), full((H, H)), full((32, H)), full((8, H)), full((1, H)),
            full((H, H)), full((1, H)), full((H, 8)), full((1, 8)),
            full((H, H)), full((1, H)), full((H, 8)),
            full((H, H)), full((H, H)), full((1, H)), full((H, H)), full((1, H)),
            full((1, 32)),
        ],
        out_specs=[
            pl.BlockSpec((RB, H), lambda i: (i, 0)),
            pl.BlockSpec((RB, 4), lambda i: (i, 0)),
        ],
        out_shape=[
            jax.ShapeDtypeStruct((N, H), jnp.float32),
            jax.ShapeDtypeStruct((N, 4), jnp.float32),
        ],
        compiler_params=pltpu.CompilerParams(
            dimension_semantics=("parallel",)),
    )(gh, gx, h, xp, *wts, offs)


def _layer(h, x, wts, maskf, batchf, rng):
    xsq = jnp.sum(x * x, axis=1)
    xr = x
    zeros1 = jnp.zeros((N, 1), jnp.float32)
    zeros2 = jnp.zeros((N, 2), jnp.float32)
    a = jnp.concatenate([xr * 2.0, xsq[:, None], zeros1, batchf[:, None], zeros2],
                        axis=1)
    bmat = jnp.zeros((8, NPAD), jnp.float32)
    bmat = bmat.at[0:3, :N].set(xr.T)
    bmat = bmat.at[4, :N].set(xsq)
    bcol = jnp.full((1, NPAD), -1.0, jnp.float32).at[0, :N].set(batchf)

    nbr = _knn_call(rng, a, bmat, bcol)

    xp4 = jnp.concatenate([x, maskf[:, None]], axis=1)
    gh, gxf = _gather_call(h, nbr.reshape(NE), xp4.reshape(N * 4))
    gx = gxf.reshape(NE, 4)

    offs = jnp.zeros((1, 32), jnp.float32).at[0, :NG].set(_OFF)
    h2, x2p = _edge_call(gh, gx, h, xp4, wts, offs)
    return h2, x2p[:, 0:3]


def kernel(h, x, We1, be1, We2, be2, Winf, binf, Wx1, bx1, Wx2,
           Wn1, bn1, Wn2, bn2, mask_ligand, batch):
    maskf = mask_ligand.astype(jnp.float32)
    batchf = batch.astype(jnp.float32)

    firsts = batch[::KRB]
    lasts = batch[KRB - 1::KRB]
    lo = jnp.searchsorted(batch, firsts, side="left").astype(jnp.int32)
    hi = jnp.searchsorted(batch, lasts, side="right").astype(jnp.int32)
    lo = (lo // 128) * 128
    rng = jnp.stack([lo, hi], axis=1)

    for l in range(L):
        winf8 = jnp.zeros((H, 8), jnp.float32).at[:, 0].set(Winf[l, :, 0])
        binf8 = jnp.zeros((1, 8), jnp.float32).at[0, 0].set(binf[l, 0])
        wx28 = jnp.zeros((H, 8), jnp.float32).at[:, 0].set(Wx2[l, :, 0])
        w1c = jnp.zeros((32, H), jnp.float32).at[0:NG, :].set(We1[l, 2 * H:2 * H + NG, :])
        w1d = jnp.zeros((8, H), jnp.float32).at[0:4, :].set(We1[l, 2 * H + NG:, :])
        wts = (
            We1[l, 0:H, :], We1[l, H:2 * H, :], w1c, w1d, be1[l][None, :],
            We2[l], be2[l][None, :], winf8, binf8,
            Wx1[l], bx1[l][None, :], wx28,
            Wn1[l, 0:H, :], Wn1[l, H:, :], bn1[l][None, :], Wn2[l], bn2[l][None, :],
        )
        h, x = _layer(h, x, wts, maskf, batchf, rng)
    return (h, x)


# speedup vs baseline: 1.5813x; 1.5813x over previous
import functools

import jax
import jax.numpy as jnp
from jax import lax
from jax.experimental import pallas as pl
from jax.experimental.pallas import tpu as pltpu
from jax.experimental.pallas import tpu_sc as plsc

N = 10000
K = 16
L = 2
H = 128
NG = 20
NE = N * K
RB = 400
NBLK = N // RB
EB = RB * K
KRB = 200
KNBLK = N // KRB
CB = 1024
NPAD = 11136

_OFF = jnp.linspace(0.0, 10.0, NG)
_COEFF = -0.5 / (float(_OFF[1] - _OFF[0]) ** 2)

_BIGI = 0x3FFFFFFF


def _knn_body(rng_ref, a_ref, b_ref, bcol_ref, nbr_ref):
    i = pl.program_id(0)
    lo = rng_ref[i, 0]
    hi = rng_ref[i, 1]
    ablk = a_ref[...]
    brow = ablk[:, 5:6]
    sqr = ablk[:, 3:4]
    rowg = lax.broadcasted_iota(jnp.int32, (KRB, 1), 0) + i * KRB
    ntiles = (hi - lo + CB - 1) // CB

    def tile_step(t, carry):
        bd, bi = carry
        start = pl.multiple_of(lo + t * CB, 128)
        bt = b_ref[:, pl.ds(start, CB)]
        bct = bcol_ref[:, pl.ds(start, CB)]
        sqc = bt[4:5, :]
        cross = jnp.dot(ablk, bt, preferred_element_type=jnp.float32)
        d = (sqr + sqc) - cross
        colg = lax.broadcasted_iota(jnp.int32, (KRB, CB), 1) + start
        valid = (brow == bct) & (colg != rowg)
        d = jnp.where(valid, d, jnp.inf)
        lane_k = lax.broadcasted_iota(jnp.int32, (KRB, K), 1)
        nbd = jnp.full((KRB, K), jnp.inf, jnp.float32)
        nbi = jnp.full((KRB, K), _BIGI, jnp.int32)
        for k in range(K):
            m = jnp.minimum(jnp.min(d, axis=1, keepdims=True),
                            jnp.min(bd, axis=1, keepdims=True))
            ct = jnp.min(jnp.where(d <= m, colg, _BIGI), axis=1, keepdims=True)
            cb = jnp.min(jnp.where(bd <= m, bi, _BIGI), axis=1, keepdims=True)
            amin = jnp.minimum(ct, cb)
            d = jnp.where(colg == amin, jnp.inf, d)
            bd = jnp.where(bi == amin, jnp.inf, bd)
            nbd = jnp.where(lane_k == k, m, nbd)
            nbi = jnp.where(lane_k == k, amin, nbi)
        return nbd, nbi

    bd0 = jnp.full((KRB, K), jnp.inf, jnp.float32)
    bi0 = jnp.full((KRB, K), _BIGI, jnp.int32)
    _, bi = lax.fori_loop(0, ntiles, tile_step, (bd0, bi0))
    nbr_ref[...] = bi


def _knn_call(rng, a, b, bcol):
    return pl.pallas_call(
        _knn_body,
        grid=(KNBLK,),
        in_specs=[
            pl.BlockSpec(memory_space=pltpu.SMEM),
            pl.BlockSpec((KRB, 8), lambda i: (i, 0)),
            pl.BlockSpec((8, NPAD), lambda i: (0, 0)),
            pl.BlockSpec((1, NPAD), lambda i: (0, 0)),
        ],
        out_specs=pl.BlockSpec((KRB, K), lambda i: (i, 0)),
        out_shape=jax.ShapeDtypeStruct((N, K), jnp.int32),
        compiler_params=pltpu.CompilerParams(
            dimension_semantics=("parallel",)),
    )(rng, a, b, bcol)


_NW = 32
_PW = NE // _NW
_CH = 104
_NCHT = 49
_NG16 = _PW // 16


def _gather_body(h_hbm, idx_hbm, xm_hbm, outh_hbm, outx_hbm,
                 idx_all, xm4, xmout, buf0, buf1, semi, sem0, sem1):
    wid = lax.axis_index("s") * 2 + lax.axis_index("c")
    base = wid * _PW
    pltpu.async_copy(idx_hbm.at[pl.ds(base, _PW)], idx_all, semi).wait()

    def off_of(c):
        return jnp.minimum(c * _CH, _PW - _CH)

    def issue(c, buf, sem):
        pltpu.async_copy(h_hbm.at[idx_all.at[pl.ds(off_of(c), _CH)]], buf, sem)

    def wait(buf, sem):
        pltpu.make_async_copy(h_hbm.at[pl.ds(0, _CH)], buf, sem).wait()

    def out(c, buf):
        pltpu.sync_copy(buf, outh_hbm.at[pl.ds(base + off_of(c), _CH)])

    issue(0, buf0, sem0)
    issue(1, buf1, sem1)

    pltpu.async_copy(xm_hbm, xm4, semi).wait()
    i4 = lax.iota(jnp.int32, 16) * 4

    def xstep(g, carry):
        goff = jnp.minimum(g * 16, _PW - 16)
        iv = idx_all[pl.ds(goff, 16)] * 4
        pos = i4 + goff * 4
        for c in range(4):
            v = plsc.load_gather(xm4, [iv + c])
            plsc.store_scatter(xmout, [pos + c], v)
        return carry

    lax.fori_loop(0, _NG16 + 1, xstep, 0)
    pltpu.sync_copy(xmout, outx_hbm.at[pl.ds(base * 4, _PW * 4)])

    def pair(cc, carry):
        c0 = 2 * cc
        wait(buf0, sem0)
        out(c0, buf0)
        issue(jnp.minimum(c0 + 2, _NCHT - 1), buf0, sem0)
        wait(buf1, sem1)
        out(c0 + 1, buf1)
        issue(jnp.minimum(c0 + 3, _NCHT - 1), buf1, sem1)
        return carry

    lax.fori_loop(0, 25, pair, 0)
    wait(buf0, sem0)
    wait(buf1, sem1)


def _gather_call(h, idx, xm4flat):
    mesh = plsc.VectorSubcoreMesh(
        core_axis_name="c", subcore_axis_name="s", num_cores=2, num_subcores=16)
    fn = pl.kernel(
        _gather_body,
        out_type=(jax.ShapeDtypeStruct((NE, H), jnp.float32),
                  jax.ShapeDtypeStruct((NE * 4,), jnp.float32)),
        mesh=mesh,
        compiler_params=pltpu.CompilerParams(needs_layout_passes=False),
        scratch_types=[
            pltpu.VMEM((_PW,), jnp.int32),
            pltpu.VMEM((N * 4,), jnp.float32),
            pltpu.VMEM((_PW * 4,), jnp.float32),
            pltpu.VMEM((_CH, H), jnp.float32),
            pltpu.VMEM((_CH, H), jnp.float32),
            pltpu.SemaphoreType.DMA,
            pltpu.SemaphoreType.DMA,
            pltpu.SemaphoreType.DMA,
        ],
    )
    return fn(h, idx, xm4flat)


def _bcast_e(v):
    c = v.shape[1]
    return jnp.broadcast_to(v[:, None, :], (RB, K, c)).reshape(EB, c)


def _segsum(v):
    c = v.shape[1]
    return jnp.sum(v.reshape(RB, K, c), axis=1)


def _edge_body(gh_ref, gx_ref, h_ref, xp_ref,
               w1a_ref, w1b_ref, w1c_ref, w1d_ref, b1_ref,
               w2_ref, b2_ref, winf_ref, binf_ref,
               wx1_ref, bx1_ref, wx2_ref,
               wn1a_ref, wn1b_ref, bn1_ref, wn2_ref, bn2_ref,
               offs_ref,
               h2_ref, x2_ref):
    f32 = jnp.float32
    hi_blk = h_ref[...]
    xi4 = xp_ref[...]
    hj = gh_ref[...]
    tj = gx_ref[...]

    lane4e = lax.broadcasted_iota(jnp.int32, (EB, 4), 1)

    xie = _bcast_e(xi4)
    rel = jnp.where(lane4e < 3, xie - tj, 0.0)
    d_sq = jnp.sum(rel * rel, axis=1, keepdims=True)
    dd = jnp.sqrt(d_sq + 1e-8)
    offs = offs_ref[...]
    dfeat = jnp.exp(_COEFF * (dd - offs) ** 2)

    msrc = tj[:, 3:4]
    mdst_n = xi4[:, 3:4]
    mdst = _bcast_e(mdst_n)
    lane8 = lax.broadcasted_iota(jnp.int32, (EB, 8), 1)
    ea = ((lane8 == 0).astype(f32) * (msrc * mdst)
          + (lane8 == 1).astype(f32) * (msrc * (1.0 - mdst))
          + (lane8 == 2).astype(f32) * ((1.0 - msrc) * mdst)
          + (lane8 == 3).astype(f32) * ((1.0 - msrc) * (1.0 - mdst)))

    pre_i = jnp.dot(hi_blk, w1a_ref[...], preferred_element_type=f32) + b1_ref[...]
    pre = (_bcast_e(pre_i)
           + jnp.dot(hj, w1b_ref[...], preferred_element_type=f32)
           + jnp.dot(dfeat, w1c_ref[...], preferred_element_type=f32)
           + jnp.dot(ea, w1d_ref[...], preferred_element_type=f32))
    t1 = pre * jax.nn.sigmoid(pre)
    t2 = jnp.dot(t1, w2_ref[...], preferred_element_type=f32) + b2_ref[...]
    mij = t2 * jax.nn.sigmoid(t2)

    zinf = jnp.dot(mij, winf_ref[...], preferred_element_type=f32) + binf_ref[...]
    zinf0 = jnp.sum(jnp.where(lane8 == 0, zinf, 0.0), axis=1, keepdims=True)
    eij = jax.nn.sigmoid(zinf0)

    mi = _segsum(mij * eij)
    u = (jnp.dot(mi, wn1a_ref[...], preferred_element_type=f32)
         + jnp.dot(hi_blk, wn1b_ref[...], preferred_element_type=f32)
         + bn1_ref[...])
    u = u * jax.nn.sigmoid(u)
    h2_ref[...] = hi_blk + jnp.dot(u, wn2_ref[...], preferred_element_type=f32) + bn2_ref[...]

    s = jnp.dot(mij, wx1_ref[...], preferred_element_type=f32) + bx1_ref[...]
    s = s * jax.nn.sigmoid(s)
    zx = jnp.dot(s, wx2_ref[...], preferred_element_type=f32)
    zx0 = jnp.sum(jnp.where(lane8 == 0, zx, 0.0), axis=1, keepdims=True)
    xm = jnp.tanh(zx0)
    delta = rel * (xm / (dd + 1.0))
    dx = _segsum(delta)
    x2_ref[...] = xi4 + dx * mdst_n


def _edge_call(gh, gx, h, xp, wts, offs):
    full = lambda shape: pl.BlockSpec(shape, lambda i: (0, 0))
    return pl.pallas_call(
        _edge_body,
        grid=(NBLK,),
        in_specs=[
            pl.BlockSpec((EB, H), lambda i: (i, 0)),
            pl.BlockSpec((EB, 4), lambda i: (i, 0)),
            pl.BlockSpec((RB, H), lambda i: (i, 0)),
            pl.BlockSpec((RB, 4), lambda i: (i, 0)),
            full((H, H)), full((H, H)), full((32, H)), full((8, H)), full((1, H)),
            full((H, H)), full((1, H)), full((H, 8)), full((1, 8)),
            full((H, H)), full((1, H)), full((H, 8)),
            full((H, H)), full((H, H)), full((1, H)), full((H, H)), full((1, H)),
            full((1, 32)),
        ],
        out_specs=[
            pl.BlockSpec((RB, H), lambda i: (i, 0)),
            pl.BlockSpec((RB, 4), lambda i: (i, 0)),
        ],
        out_shape=[
            jax.ShapeDtypeStruct((N, H), jnp.float32),
            jax.ShapeDtypeStruct((N, 4), jnp.float32),
        ],
        compiler_params=pltpu.CompilerParams(
            dimension_semantics=("parallel",)),
    )(gh, gx, h, xp, *wts, offs)


def _layer(h, x, wts, maskf, batchf, rng):
    xsq = jnp.sum(x * x, axis=1)
    xr = x
    zeros1 = jnp.zeros((N, 1), jnp.float32)
    zeros2 = jnp.zeros((N, 2), jnp.float32)
    a = jnp.concatenate([xr * 2.0, xsq[:, None], zeros1, batchf[:, None], zeros2],
                        axis=1)
    bmat = jnp.zeros((8, NPAD), jnp.float32)
    bmat = bmat.at[0:3, :N].set(xr.T)
    bmat = bmat.at[4, :N].set(xsq)
    bcol = jnp.full((1, NPAD), -1.0, jnp.float32).at[0, :N].set(batchf)

    nbr = _knn_call(rng, a, bmat, bcol)

    xp4 = jnp.concatenate([x, maskf[:, None]], axis=1)
    gh, gxf = _gather_call(h, nbr.reshape(NE), xp4.reshape(N * 4))
    gx = gxf.reshape(NE, 4)

    offs = jnp.zeros((1, 32), jnp.float32).at[0, :NG].set(_OFF)
    h2, x2p = _edge_call(gh, gx, h, xp4, wts, offs)
    return h2, x2p[:, 0:3]


def kernel(h, x, We1, be1, We2, be2, Winf, binf, Wx1, bx1, Wx2,
           Wn1, bn1, Wn2, bn2, mask_ligand, batch):
    maskf = mask_ligand.astype(jnp.float32)
    batchf = batch.astype(jnp.float32)

    firsts = batch[::KRB]
    lasts = batch[KRB - 1::KRB]
    lo = jnp.searchsorted(batch, firsts, side="left").astype(jnp.int32)
    hi = jnp.searchsorted(batch, lasts, side="right").astype(jnp.int32)
    lo = (lo // 128) * 128
    rng = jnp.stack([lo, hi], axis=1)

    for l in range(L):
        winf8 = jnp.zeros((H, 8), jnp.float32).at[:, 0].set(Winf[l, :, 0])
        binf8 = jnp.zeros((1, 8), jnp.float32).at[0, 0].set(binf[l, 0])
        wx28 = jnp.zeros((H, 8), jnp.float32).at[:, 0].set(Wx2[l, :, 0])
        w1c = jnp.zeros((32, H), jnp.float32).at[0:NG, :].set(We1[l, 2 * H:2 * H + NG, :])
        w1d = jnp.zeros((8, H), jnp.float32).at[0:4, :].set(We1[l, 2 * H + NG:, :])
        wts = (
            We1[l, 0:H, :], We1[l, H:2 * H, :], w1c, w1d, be1[l][None, :],
            We2[l], be2[l][None, :], winf8, binf8,
            Wx1[l], bx1[l][None, :], wx28,
            Wn1[l, 0:H, :], Wn1[l, H:, :], bn1[l][None, :], Wn2[l], bn2[l][None, :],
        )
        h, x = _layer(h, x, wts, maskf, batchf, rng)
    return (h, x)
